# baseline (speedup 1.0000x reference)
import functools

import jax
import jax.numpy as jnp
from jax.experimental import pallas as pl
from jax.experimental.pallas import tpu as pltpu

B, T, D, H, NT = 2, 2048, 768, 12, 8
DF = 4 * D
DH = D // H
BT = B * T
TILE = DF // NT

_F32 = jnp.float32


def _ln(xb, g, b, eps=1e-5):
    m = jnp.mean(xb, axis=-1, keepdims=True)
    v = jnp.mean((xb - m) ** 2, axis=-1, keepdims=True)
    return (xb - m) / jnp.sqrt(v + eps) * g + b


def _k1_body(x_ref, g_ref, b_ref, wt_ref, pb_ref, qkv_ref):
    normed = _ln(x_ref[...], g_ref[...], b_ref[...])
    qkv_ref[...] = (
        jnp.dot(normed, wt_ref[...], preferred_element_type=_F32) + pb_ref[...]
    )


def _k1(x2d, ln1_g, ln1_b, wT, pb, blk=512):
    return pl.pallas_call(
        _k1_body,
        grid=(BT // blk,),
        in_specs=[
            pl.BlockSpec((blk, D), lambda i: (i, 0)),
            pl.BlockSpec((1, D), lambda i: (0, 0)),
            pl.BlockSpec((1, D), lambda i: (0, 0)),
            pl.BlockSpec((D, 3 * D), lambda i: (0, 0)),
            pl.BlockSpec((1, 3 * D), lambda i: (0, 0)),
        ],
        out_specs=pl.BlockSpec((blk, 3 * D), lambda i: (i, 0)),
        out_shape=jax.ShapeDtypeStruct((BT, 3 * D), _F32),
    )(x2d, ln1_g, ln1_b, wT, pb)


def _k2_body(q_ref, k_ref, v_ref, o_ref, *, bq):
    i = pl.program_id(2)
    q = q_ref[0, 0]
    k = k_ref[0, 0]
    v = v_ref[0, 0]
    s = jax.lax.dot_general(
        q, k, (((1,), (1,)), ((), ())), preferred_element_type=_F32
    ) / jnp.sqrt(jnp.asarray(DH, _F32))
    row = i * bq + jax.lax.broadcasted_iota(jnp.int32, (bq, T), 0)
    col = jax.lax.broadcasted_iota(jnp.int32, (bq, T), 1)
    s = jnp.where(col <= row, s, -jnp.inf)
    m = jnp.max(s, axis=-1, keepdims=True)
    e = jnp.exp(s - m)
    p = e / jnp.sum(e, axis=-1, keepdims=True)
    o_ref[0, 0] = jnp.dot(p, v, preferred_element_type=_F32)


def _k2(q4, k4, v4, bq=512):
    return pl.pallas_call(
        functools.partial(_k2_body, bq=bq),
        grid=(B, H, T // bq),
        in_specs=[
            pl.BlockSpec((1, 1, bq, DH), lambda b, h, i: (b, h, i, 0)),
            pl.BlockSpec((1, 1, T, DH), lambda b, h, i: (b, h, 0, 0)),
            pl.BlockSpec((1, 1, T, DH), lambda b, h, i: (b, h, 0, 0)),
        ],
        out_specs=pl.BlockSpec((1, 1, bq, DH), lambda b, h, i: (b, h, i, 0)),
        out_shape=jax.ShapeDtypeStruct((B, H, T, DH), _F32),
    )(q4, k4, v4)


def _k3_body(ctx_ref, x_ref, owt_ref, ob_ref, g2_ref, b2_ref, gwt_ref,
             gb_ref, x1_ref, n2_ref, gate_ref, eidx_ref):
    attn = jnp.dot(ctx_ref[...], owt_ref[...], preferred_element_type=_F32)
    x1 = x_ref[...] + (attn + ob_ref[...])
    x1_ref[...] = x1
    n2 = _ln(x1, g2_ref[...], b2_ref[...])
    n2_ref[...] = n2
    logits = jnp.dot(n2, gwt_ref[...], preferred_element_type=_F32) + gb_ref[...]
    m = jnp.max(logits, axis=-1, keepdims=True)
    colid = jax.lax.broadcasted_iota(jnp.int32, logits.shape, 1)
    idx = jnp.min(jnp.where(logits == m, colid, NT), axis=-1, keepdims=True)
    hard = (colid == idx).astype(_F32)
    gate_ref[...] = logits + (hard - logits)
    eidx_ref[...] = idx


def _k3(ctx2d, x2d, owT, ob, ln2_g, ln2_b, gwT, gb, blk=512):
    return pl.pallas_call(
        _k3_body,
        grid=(BT // blk,),
        in_specs=[
            pl.BlockSpec((blk, D), lambda i: (i, 0)),
            pl.BlockSpec((blk, D), lambda i: (i, 0)),
            pl.BlockSpec((D, D), lambda i: (0, 0)),
            pl.BlockSpec((1, D), lambda i: (0, 0)),
            pl.BlockSpec((1, D), lambda i: (0, 0)),
            pl.BlockSpec((1, D), lambda i: (0, 0)),
            pl.BlockSpec((D, NT), lambda i: (0, 0)),
            pl.BlockSpec((1, NT), lambda i: (0, 0)),
        ],
        out_specs=[
            pl.BlockSpec((blk, D), lambda i: (i, 0)),
            pl.BlockSpec((blk, D), lambda i: (i, 0)),
            pl.BlockSpec((blk, NT), lambda i: (i, 0)),
            pl.BlockSpec((blk, 1), lambda i: (i, 0)),
        ],
        out_shape=[
            jax.ShapeDtypeStruct((BT, D), _F32),
            jax.ShapeDtypeStruct((BT, D), _F32),
            jax.ShapeDtypeStruct((BT, NT), _F32),
            jax.ShapeDtypeStruct((BT, 1), jnp.int32),
        ],
    )(ctx2d, x2d, owT, ob, ln2_g, ln2_b, gwT, gb)


def _k4_body(n2_ref, x1_ref, eidx_ref, uwt_ref, ub_ref, dwt_ref, db_ref,
             o_ref):
    h = jnp.dot(n2_ref[...], uwt_ref[...], preferred_element_type=_F32)
    h = h + ub_ref[...]
    blk = h.shape[0]
    grp = jax.lax.broadcasted_iota(jnp.int32, (blk, DF), 1) // TILE
    mask = (grp == eidx_ref[...]).astype(_F32)
    h = jnp.maximum(h * mask, 0.0)
    out = jnp.dot(h, dwt_ref[...], preferred_element_type=_F32) + db_ref[...]
    o_ref[...] = x1_ref[...] + out


def _k4(n2, x1, eidx, uwT, ub, dwT, db, blk=256):
    return pl.pallas_call(
        _k4_body,
        grid=(BT // blk,),
        in_specs=[
            pl.BlockSpec((blk, D), lambda i: (i, 0)),
            pl.BlockSpec((blk, D), lambda i: (i, 0)),
            pl.BlockSpec((blk, 1), lambda i: (i, 0)),
            pl.BlockSpec((D, DF), lambda i: (0, 0)),
            pl.BlockSpec((1, DF), lambda i: (0, 0)),
            pl.BlockSpec((DF, D), lambda i: (0, 0)),
            pl.BlockSpec((1, D), lambda i: (0, 0)),
        ],
        out_specs=pl.BlockSpec((blk, D), lambda i: (i, 0)),
        out_shape=jax.ShapeDtypeStruct((BT, D), _F32),
    )(n2, x1, eidx, uwT, ub, dwT, db)


def kernel(x, ln1_g, ln1_b, ln2_g, ln2_b, in_proj_w, in_proj_b, out_proj_w,
           out_proj_b, gate_w, gate_b, up_w, up_b, down_w, down_b):
    x2d = x.reshape(BT, D)
    qkv = _k1(x2d, ln1_g.reshape(1, D), ln1_b.reshape(1, D),
              in_proj_w.T, in_proj_b.reshape(1, 3 * D))
    q, k, v = jnp.split(qkv, 3, axis=-1)

    def _heads(t):
        return t.reshape(B, T, H, DH).transpose(0, 2, 1, 3)

    ctx4 = _k2(_heads(q), _heads(k), _heads(v))
    ctx2d = ctx4.transpose(0, 2, 1, 3).reshape(BT, D)
    x1, n2, gate, eidx = _k3(
        ctx2d, x2d, out_proj_w.T, out_proj_b.reshape(1, D),
        ln2_g.reshape(1, D), ln2_b.reshape(1, D),
        gate_w.T, gate_b.reshape(1, NT))
    xo = _k4(n2, x1, eidx, up_w.T, up_b.reshape(1, DF),
             down_w.T, down_b.reshape(1, D))
    return xo.reshape(B, T, D), gate.reshape(B, T, NT)
